# baseline (speedup 1.0000x reference)
import functools

import jax
import jax.numpy as jnp
from jax import lax
from jax.experimental import pallas as pl
from jax.experimental.pallas import tpu as pltpu
from jax.experimental.pallas import tpu_sc as plsc

N_NODES = 10000
N_EDGES = 320000
D = 128

NC = 2
NS = 16
C = 128
TCH = 160
K0 = 160
K1 = TCH - K0
P = 32
NB = 2
EPT = TCH * C
E_PAD = NS * EPT
ACC_ROWS = 10112
ZROWS = ACC_ROWS // NS


def _sc_aggregate(x, src3, dst3, zeros):
    mesh = plsc.VectorSubcoreMesh(
        core_axis_name="c", subcore_axis_name="s", num_cores=NC, num_subcores=NS
    )

    @functools.partial(
        pl.kernel,
        out_type=jax.ShapeDtypeStruct((NC, ACC_ROWS, D), jnp.float32),
        mesh=mesh,
        scratch_types=[
            pltpu.VMEM_SHARED((ACC_ROWS, D), jnp.float32),
            pltpu.VMEM((P, C), jnp.int32),
            pltpu.VMEM((P, C), jnp.int32),
        ]
        + [pltpu.VMEM((C, D), jnp.float32) for _ in range(NB)]
        + [pltpu.SemaphoreType.DMA for _ in range(NB)],
    )
    def agg_kernel(x_hbm, src_hbm, dst_hbm, z_hbm, out_hbm, acc, src_v, dst_v, *ring):
        rows = ring[:NB]
        gsem = ring[NB:]
        c = lax.axis_index("c")
        s = lax.axis_index("s")

        pltpu.sync_copy(z_hbm.at[pl.ds(s * ZROWS, ZROWS)], acc.at[pl.ds(s * ZROWS, ZROWS)])
        plsc.subcore_barrier()

        lo = jnp.where(c == 0, 0, K0)
        nph = jnp.where(c == 0, K0 // P, K1 // P)

        def phase_body(ph, carry):
            base = pl.multiple_of(lo + ph * P, P)
            pltpu.sync_copy(src_hbm.at[s, pl.ds(base, P)], src_v)
            pltpu.sync_copy(dst_hbm.at[s, pl.ds(base, P)], dst_v)

            for b in range(NB):
                pltpu.async_copy(x_hbm.at[src_v.at[b]], rows[b], gsem[b])

            def body(i, carry2):
                j0 = i * NB
                for b in range(NB):
                    j = j0 + b
                    pltpu.make_async_copy(x_hbm.at[src_v.at[j]], rows[b], gsem[b]).wait()
                    pltpu.sync_copy(rows[b], acc.at[dst_v.at[j]], add=True)
                    pltpu.async_copy(x_hbm.at[src_v.at[j + NB]], rows[b], gsem[b])
                return carry2

            lax.fori_loop(0, P // NB - 1, body, 0, unroll=False)

            for b in range(NB):
                j = P - NB + b
                pltpu.make_async_copy(x_hbm.at[src_v.at[j]], rows[b], gsem[b]).wait()
                pltpu.sync_copy(rows[b], acc.at[dst_v.at[j]], add=True)
            return carry

        lax.fori_loop(0, nph, phase_body, 0, unroll=False)

        plsc.subcore_barrier()
        pltpu.sync_copy(
            acc.at[pl.ds(s * ZROWS, ZROWS)],
            out_hbm.at[c, pl.ds(s * ZROWS, ZROWS)],
        )

    return agg_kernel(x, src3, dst3, zeros)


def _mlp_body(x_ref, p_ref, w1_ref, b1_ref, g_ref, bt_ref, w2_ref, b2_ref, out_ref):
    h = x_ref[...] + p_ref[0, :N_NODES] + p_ref[1, :N_NODES]
    h1 = jnp.dot(h, w1_ref[...].T, preferred_element_type=jnp.float32) + b1_ref[...]
    mean = jnp.mean(h1, axis=0, keepdims=True)
    var = jnp.mean((h1 - mean) ** 2, axis=0, keepdims=True)
    hn = (h1 - mean) * lax.rsqrt(var + 1e-5) * g_ref[...] + bt_ref[...]
    hr = jnp.maximum(hn, 0.0)
    h2 = jnp.dot(hr, w2_ref[...].T, preferred_element_type=jnp.float32) + b2_ref[...]
    out_ref[...] = jnp.maximum(h2, 0.0)


def kernel(x, edge_index, W1, b1, gamma, beta, W2, b2):
    src = edge_index[0].astype(jnp.int32)
    dst = edge_index[1].astype(jnp.int32)
    pad = E_PAD - N_EDGES
    src3 = jnp.concatenate([src, jnp.zeros((pad,), jnp.int32)]).reshape(NS, TCH, C)
    dst3 = jnp.concatenate([dst, jnp.full((pad,), N_NODES, jnp.int32)]).reshape(NS, TCH, C)
    zeros = jnp.zeros((ACC_ROWS, D), jnp.float32)

    partials = _sc_aggregate(x, src3, dst3, zeros)

    return pl.pallas_call(
        _mlp_body,
        out_shape=jax.ShapeDtypeStruct((N_NODES, D), jnp.float32),
    )(
        x,
        partials,
        W1,
        b1.reshape(1, D),
        gamma.reshape(1, D),
        beta.reshape(1, D),
        W2,
        b2.reshape(1, D),
    )


# speedup vs baseline: 1.3674x; 1.3674x over previous
import functools

import jax
import jax.numpy as jnp
from jax import lax
from jax.experimental import pallas as pl
from jax.experimental.pallas import tpu as pltpu
from jax.experimental.pallas import tpu_sc as plsc

N_NODES = 10000
N_EDGES = 320000
D = 128

NC = 2
NS = 16
C = 128
TCH = 160
K0 = 128
K1 = TCH - K0
P = 32
NB = 2
EPT = TCH * C
E_PAD = NS * EPT
ACC_ROWS = 10112
ZROWS = ACC_ROWS // NS


def _sc_aggregate(x, src3, dst3, zeros):
    mesh = plsc.VectorSubcoreMesh(
        core_axis_name="c", subcore_axis_name="s", num_cores=NC, num_subcores=NS
    )

    @functools.partial(
        pl.kernel,
        out_type=jax.ShapeDtypeStruct((NC, ACC_ROWS, D), jnp.float32),
        mesh=mesh,
        scratch_types=[
            pltpu.VMEM_SHARED((ACC_ROWS, D), jnp.float32),
            pltpu.VMEM((P, C), jnp.int32),
            pltpu.VMEM((P, C), jnp.int32),
        ]
        + [pltpu.VMEM((C, D), jnp.float32) for _ in range(NB)]
        + [pltpu.SemaphoreType.DMA for _ in range(2 * NB)],
    )
    def agg_kernel(x_hbm, src_hbm, dst_hbm, z_hbm, out_hbm, acc, src_v, dst_v, *ring):
        rows = ring[:NB]
        gsem = ring[NB:]

        H = C // 2

        def start_gather(j, b):
            pltpu.async_copy(x_hbm.at[src_v.at[j, pl.ds(0, H)]], rows[b].at[pl.ds(0, H)], gsem[2 * b])
            pltpu.async_copy(x_hbm.at[src_v.at[j, pl.ds(H, H)]], rows[b].at[pl.ds(H, H)], gsem[2 * b + 1])

        def wait_gather(j, b):
            pltpu.make_async_copy(x_hbm.at[src_v.at[j, pl.ds(0, H)]], rows[b].at[pl.ds(0, H)], gsem[2 * b]).wait()
            pltpu.make_async_copy(x_hbm.at[src_v.at[j, pl.ds(H, H)]], rows[b].at[pl.ds(H, H)], gsem[2 * b + 1]).wait()
        c = lax.axis_index("c")
        s = lax.axis_index("s")

        pltpu.sync_copy(z_hbm.at[pl.ds(s * ZROWS, ZROWS)], acc.at[pl.ds(s * ZROWS, ZROWS)])
        plsc.subcore_barrier()

        lo = jnp.where(c == 0, 0, K0)
        nph = jnp.where(c == 0, K0 // P, K1 // P)

        def phase_body(ph, carry):
            base = pl.multiple_of(lo + ph * P, P)
            pltpu.sync_copy(src_hbm.at[s, pl.ds(base, P)], src_v)
            pltpu.sync_copy(dst_hbm.at[s, pl.ds(base, P)], dst_v)

            for b in range(NB):
                start_gather(b, b)

            def body(i, carry2):
                j0 = i * NB
                for b in range(NB):
                    j = j0 + b
                    wait_gather(j, b)
                    pltpu.sync_copy(rows[b], acc.at[dst_v.at[j]], add=True)
                    start_gather(j + NB, b)
                return carry2

            lax.fori_loop(0, P // NB - 1, body, 0, unroll=False)

            for b in range(NB):
                j = P - NB + b
                wait_gather(j, b)
                pltpu.sync_copy(rows[b], acc.at[dst_v.at[j]], add=True)
            return carry

        lax.fori_loop(0, nph, phase_body, 0, unroll=False)

        plsc.subcore_barrier()
        pltpu.sync_copy(
            acc.at[pl.ds(s * ZROWS, ZROWS)],
            out_hbm.at[c, pl.ds(s * ZROWS, ZROWS)],
        )

    return agg_kernel(x, src3, dst3, zeros)


def _mlp_body(x_ref, p_ref, w1_ref, b1_ref, g_ref, bt_ref, w2_ref, b2_ref, out_ref):
    h = x_ref[...] + p_ref[0, :N_NODES] + p_ref[1, :N_NODES]
    h1 = jnp.dot(h, w1_ref[...].T, preferred_element_type=jnp.float32) + b1_ref[...]
    mean = jnp.mean(h1, axis=0, keepdims=True)
    var = jnp.mean((h1 - mean) ** 2, axis=0, keepdims=True)
    hn = (h1 - mean) * lax.rsqrt(var + 1e-5) * g_ref[...] + bt_ref[...]
    hr = jnp.maximum(hn, 0.0)
    h2 = jnp.dot(hr, w2_ref[...].T, preferred_element_type=jnp.float32) + b2_ref[...]
    out_ref[...] = jnp.maximum(h2, 0.0)


def kernel(x, edge_index, W1, b1, gamma, beta, W2, b2):
    src = edge_index[0].astype(jnp.int32)
    dst = edge_index[1].astype(jnp.int32)
    pad = E_PAD - N_EDGES
    src3 = jnp.concatenate([src, jnp.zeros((pad,), jnp.int32)]).reshape(NS, TCH, C)
    dst3 = jnp.concatenate([dst, jnp.full((pad,), N_NODES, jnp.int32)]).reshape(NS, TCH, C)
    zeros = jnp.zeros((ACC_ROWS, D), jnp.float32)

    partials = _sc_aggregate(x, src3, dst3, zeros)

    return pl.pallas_call(
        _mlp_body,
        out_shape=jax.ShapeDtypeStruct((N_NODES, D), jnp.float32),
    )(
        x,
        partials,
        W1,
        b1.reshape(1, D),
        gamma.reshape(1, D),
        beta.reshape(1, D),
        W2,
        b2.reshape(1, D),
    )


# speedup vs baseline: 1.3732x; 1.0042x over previous
import functools

import jax
import jax.numpy as jnp
from jax import lax
from jax.experimental import pallas as pl
from jax.experimental.pallas import tpu as pltpu
from jax.experimental.pallas import tpu_sc as plsc

N_NODES = 10000
N_EDGES = 320000
D = 128

NC = 2
NS = 16
C = 128
TCH = 160
K0 = 128
K1 = TCH - K0
P = 32
NB = 2
EPT = TCH * C
E_PAD = NS * EPT
ACC_ROWS = 10112
ZROWS = ACC_ROWS // NS


def _sc_aggregate(x, src3, dst3):
    mesh = plsc.VectorSubcoreMesh(
        core_axis_name="c", subcore_axis_name="s", num_cores=NC, num_subcores=NS
    )

    @functools.partial(
        pl.kernel,
        out_type=jax.ShapeDtypeStruct((NC, ACC_ROWS, D), jnp.float32),
        mesh=mesh,
        scratch_types=[
            pltpu.VMEM_SHARED((ACC_ROWS, D), jnp.float32),
            pltpu.VMEM((P, C), jnp.int32),
            pltpu.VMEM((P, C), jnp.int32),
        ]
        + [pltpu.VMEM((C, D), jnp.float32) for _ in range(NB)]
        + [pltpu.SemaphoreType.DMA for _ in range(NB)],
    )
    def agg_kernel(x_hbm, src_hbm, dst_hbm, out_hbm, acc, src_v, dst_v, *ring):
        rows = ring[:NB]
        gsem = ring[NB:]
        c = lax.axis_index("c")
        s = lax.axis_index("s")

        zvec = jnp.zeros((16,), jnp.float32)

        def zfill(i, carry):
            rows[0][i // 8, pl.ds((i % 8) * 16, 16)] = zvec
            return carry

        lax.fori_loop(0, C * D // 16, zfill, 0, unroll=False)
        for k in range(ZROWS // C):
            pltpu.sync_copy(rows[0], acc.at[pl.ds(s * ZROWS + k * C, C)])
        rem = ZROWS % C
        if rem:
            pltpu.sync_copy(
                rows[0].at[pl.ds(0, rem)],
                acc.at[pl.ds(s * ZROWS + (ZROWS // C) * C, rem)],
            )
        plsc.subcore_barrier()

        lo = jnp.where(c == 0, 0, K0)
        nph = jnp.where(c == 0, K0 // P, K1 // P)

        def phase_body(ph, carry):
            base = pl.multiple_of(lo + ph * P, P)
            pltpu.sync_copy(src_hbm.at[s, pl.ds(base, P)], src_v)
            pltpu.sync_copy(dst_hbm.at[s, pl.ds(base, P)], dst_v)

            for b in range(NB):
                pltpu.async_copy(x_hbm.at[src_v.at[b]], rows[b], gsem[b])

            def body(i, carry2):
                j0 = i * NB
                for b in range(NB):
                    j = j0 + b
                    pltpu.make_async_copy(x_hbm.at[src_v.at[j]], rows[b], gsem[b]).wait()
                    pltpu.sync_copy(rows[b], acc.at[dst_v.at[j]], add=True)
                    pltpu.async_copy(x_hbm.at[src_v.at[j + NB]], rows[b], gsem[b])
                return carry2

            lax.fori_loop(0, P // NB - 1, body, 0, unroll=False)

            for b in range(NB):
                j = P - NB + b
                pltpu.make_async_copy(x_hbm.at[src_v.at[j]], rows[b], gsem[b]).wait()
                pltpu.sync_copy(rows[b], acc.at[dst_v.at[j]], add=True)
            return carry

        lax.fori_loop(0, nph, phase_body, 0, unroll=False)

        plsc.subcore_barrier()
        pltpu.sync_copy(
            acc.at[pl.ds(s * ZROWS, ZROWS)],
            out_hbm.at[c, pl.ds(s * ZROWS, ZROWS)],
        )

    return agg_kernel(x, src3, dst3)


def _mlp_body(x_ref, p_ref, w1_ref, b1_ref, g_ref, bt_ref, w2_ref, b2_ref, out_ref):
    h = x_ref[...] + p_ref[0, :N_NODES] + p_ref[1, :N_NODES]
    h1 = jnp.dot(h, w1_ref[...].T, preferred_element_type=jnp.float32) + b1_ref[...]
    mean = jnp.mean(h1, axis=0, keepdims=True)
    var = jnp.mean((h1 - mean) ** 2, axis=0, keepdims=True)
    hn = (h1 - mean) * lax.rsqrt(var + 1e-5) * g_ref[...] + bt_ref[...]
    hr = jnp.maximum(hn, 0.0)
    h2 = jnp.dot(hr, w2_ref[...].T, preferred_element_type=jnp.float32) + b2_ref[...]
    out_ref[...] = jnp.maximum(h2, 0.0)


def kernel(x, edge_index, W1, b1, gamma, beta, W2, b2):
    src = edge_index[0].astype(jnp.int32)
    dst = edge_index[1].astype(jnp.int32)
    pad = E_PAD - N_EDGES
    src3 = jnp.concatenate([src, jnp.zeros((pad,), jnp.int32)]).reshape(NS, TCH, C)
    dst3 = jnp.concatenate([dst, jnp.full((pad,), N_NODES, jnp.int32)]).reshape(NS, TCH, C)

    partials = _sc_aggregate(x, src3, dst3)

    return pl.pallas_call(
        _mlp_body,
        out_shape=jax.ShapeDtypeStruct((N_NODES, D), jnp.float32),
    )(
        x,
        partials,
        W1,
        b1.reshape(1, D),
        gamma.reshape(1, D),
        beta.reshape(1, D),
        W2,
        b2.reshape(1, D),
    )
